# baseline (speedup 1.0000x reference)
import math

import jax
import jax.numpy as jnp
from jax import lax
from jax.experimental import pallas as pl
from jax.experimental.pallas import tpu as pltpu
from jax.experimental.pallas import tpu_sc as plsc

_SIZE = 100000
_N = 1024
_SMOOTHING = 0.1
_CONF = 1.0 - _SMOOTHING
_FILL = _SMOOTHING / (_SIZE - 2)
_C_ROW = (_SIZE - 2) * _FILL * math.log(_FILL) + _CONF * math.log(_CONF)

_NC = 2
_NS = 16
_LANES = 16
_NW = _NC * _NS
_B = _N // _NW

_BR = 32


def _sc_body(xr, tgt, out, tgt_v, idx_v, idx0_v, xt_v, x0_v, r_v, sem):
    wid = lax.axis_index("s") * _NC + lax.axis_index("c")
    base = wid * _B
    pltpu.sync_copy(tgt.at[pl.ds(base, _B)], tgt_v)
    for k in range(_B // _LANES):
        t = tgt_v[pl.ds(k * _LANES, _LANES)]
        i = base + k * _LANES + lax.iota(jnp.int32, _LANES)
        idx_v[pl.ds(k * _LANES, _LANES)] = i * _SIZE + t
        idx0_v[pl.ds(k * _LANES, _LANES)] = i * _SIZE
    pltpu.async_copy(xr.at[idx_v], xt_v, sem).wait()
    pltpu.async_copy(xr.at[idx0_v], x0_v, sem).wait()
    for k in range(_B // _LANES):
        t = tgt_v[pl.ds(k * _LANES, _LANES)]
        xt = xt_v[pl.ds(k * _LANES, _LANES)]
        x0 = x0_v[pl.ds(k * _LANES, _LANES)]
        r = jnp.where(
            t != 0,
            jnp.float32(_C_ROW)
            + jnp.float32(_FILL) * x0
            - jnp.float32(_CONF - _FILL) * xt,
            jnp.float32(0.0),
        )
        r_v[pl.ds(k * _LANES, _LANES)] = r
    pltpu.sync_copy(r_v, out.at[pl.ds(base, _B)])


def _sc_gather(xr, target):
    return pl.kernel(
        _sc_body,
        out_type=jax.ShapeDtypeStruct((_N,), jnp.float32),
        mesh=plsc.VectorSubcoreMesh(core_axis_name="c", subcore_axis_name="s"),
        scratch_types=[
            pltpu.VMEM((_B,), jnp.int32),
            pltpu.VMEM((_B,), jnp.int32),
            pltpu.VMEM((_B,), jnp.int32),
            pltpu.VMEM((_B,), jnp.float32),
            pltpu.VMEM((_B,), jnp.float32),
            pltpu.VMEM((_B,), jnp.float32),
            pltpu.SemaphoreType.DMA,
        ],
    )(xr, target)


def _tc_body(tgt_ref, r_ref, x_ref, out_ref, acc_ref):
    j = pl.program_id(0)

    @pl.when(j == 0)
    def _init():
        acc_ref[0] = jnp.float32(0.0)

    mask = tgt_ref[...] != 0
    acc_ref[0] += jnp.sum(jnp.where(mask, x_ref[...], jnp.float32(0.0)))

    @pl.when(j == pl.num_programs(0) - 1)
    def _finish():
        loss = jnp.sum(r_ref[...]) - jnp.float32(_FILL) * acc_ref[0]
        out_ref[...] = jnp.broadcast_to(loss, (1, 1))


def _tc_reduce(x, tgt2d, r2d):
    grid = (_N // _BR,)
    return pl.pallas_call(
        _tc_body,
        grid=grid,
        in_specs=[
            pl.BlockSpec((_BR, 1), lambda j: (j, 0)),
            pl.BlockSpec((_N, 1), lambda j: (0, 0)),
            pl.BlockSpec((_BR, _SIZE), lambda j: (j, 0)),
        ],
        out_specs=pl.BlockSpec((1, 1), lambda j: (0, 0)),
        out_shape=jax.ShapeDtypeStruct((1, 1), jnp.float32),
        scratch_shapes=[pltpu.SMEM((1,), jnp.float32)],
        compiler_params=pltpu.CompilerParams(
            dimension_semantics=("arbitrary",),
        ),
    )(tgt2d, r2d, x)


def kernel(x, target):
    xr = x.reshape(_N * _SIZE)
    r = _sc_gather(xr, target)
    out = _tc_reduce(x, target.reshape(_N, 1), r.reshape(_N, 1))
    return out[0, 0]


# speedup vs baseline: 1.0098x; 1.0098x over previous
import math

import jax
import jax.numpy as jnp
from jax import lax
from jax.experimental import pallas as pl
from jax.experimental.pallas import tpu as pltpu
from jax.experimental.pallas import tpu_sc as plsc

_SIZE = 100000
_N = 1024
_SMOOTHING = 0.1
_CONF = 1.0 - _SMOOTHING
_FILL = _SMOOTHING / (_SIZE - 2)
_C_ROW = (_SIZE - 2) * _FILL * math.log(_FILL) + _CONF * math.log(_CONF)

_NC = 2
_NS = 16
_LANES = 16
_NW = _NC * _NS
_B = _N // _NW

_BR = 32
_NBUF = 4
_NSTEPS = _N // _BR


def _sc_body(xr, tgt, out, tgt_v, idx_v, idx0_v, xt_v, x0_v, r_v, sem):
    wid = lax.axis_index("s") * _NC + lax.axis_index("c")
    base = wid * _B
    pltpu.sync_copy(tgt.at[pl.ds(base, _B)], tgt_v)
    for k in range(_B // _LANES):
        t = tgt_v[pl.ds(k * _LANES, _LANES)]
        i = base + k * _LANES + lax.iota(jnp.int32, _LANES)
        idx_v[pl.ds(k * _LANES, _LANES)] = i * _SIZE + t
        idx0_v[pl.ds(k * _LANES, _LANES)] = i * _SIZE
    pltpu.async_copy(xr.at[idx_v], xt_v, sem).wait()
    pltpu.async_copy(xr.at[idx0_v], x0_v, sem).wait()
    for k in range(_B // _LANES):
        t = tgt_v[pl.ds(k * _LANES, _LANES)]
        xt = xt_v[pl.ds(k * _LANES, _LANES)]
        x0 = x0_v[pl.ds(k * _LANES, _LANES)]
        r = jnp.where(
            t != 0,
            jnp.float32(_C_ROW)
            + jnp.float32(_FILL) * x0
            - jnp.float32(_CONF - _FILL) * xt,
            jnp.float32(0.0),
        )
        r_v[pl.ds(k * _LANES, _LANES)] = r
    pltpu.sync_copy(r_v, out.at[pl.ds(base, _B)])


def _sc_gather(xr, target):
    return pl.kernel(
        _sc_body,
        out_type=jax.ShapeDtypeStruct((_N,), jnp.float32),
        mesh=plsc.VectorSubcoreMesh(core_axis_name="c", subcore_axis_name="s"),
        scratch_types=[
            pltpu.VMEM((_B,), jnp.int32),
            pltpu.VMEM((_B,), jnp.int32),
            pltpu.VMEM((_B,), jnp.int32),
            pltpu.VMEM((_B,), jnp.float32),
            pltpu.VMEM((_B,), jnp.float32),
            pltpu.VMEM((_B,), jnp.float32),
            pltpu.SemaphoreType.DMA,
        ],
    )(xr, target)


def _start_copy(x_ref, buf_ref, sems, step):
    slot = lax.rem(step, _NBUF)
    pltpu.make_async_copy(
        x_ref.at[pl.ds(step * _BR, _BR), :],
        buf_ref.at[slot],
        sems.at[slot],
    ).start()


def _tc_body(tgt_ref, r_ref, x_ref, out_ref, buf_ref, sems, acc_ref):
    j = pl.program_id(0)

    @pl.when(j == 0)
    def _init():
        acc_ref[0] = jnp.float32(0.0)
        for k in range(_NBUF - 1):
            _start_copy(x_ref, buf_ref, sems, k)

    @pl.when(j + _NBUF - 1 < _NSTEPS)
    def _prefetch():
        _start_copy(x_ref, buf_ref, sems, j + _NBUF - 1)

    slot = lax.rem(j, _NBUF)
    pltpu.make_async_copy(
        x_ref.at[pl.ds(j * _BR, _BR), :], buf_ref.at[slot], sems.at[slot]
    ).wait()
    mask = tgt_ref[pl.ds(j * _BR, _BR), :] != 0
    acc_ref[0] += jnp.sum(jnp.where(mask, buf_ref[slot], jnp.float32(0.0)))

    @pl.when(j == _NSTEPS - 1)
    def _finish():
        loss = jnp.sum(r_ref[...]) - jnp.float32(_FILL) * acc_ref[0]
        out_ref[...] = jnp.broadcast_to(loss, (1, 1))


def _tc_reduce(x, tgt2d, r2d):
    return pl.pallas_call(
        _tc_body,
        grid=(_NSTEPS,),
        in_specs=[
            pl.BlockSpec((_N, 1), lambda j: (0, 0)),
            pl.BlockSpec((_N, 1), lambda j: (0, 0)),
            pl.BlockSpec(memory_space=pl.ANY),
        ],
        out_specs=pl.BlockSpec((1, 1), lambda j: (0, 0)),
        out_shape=jax.ShapeDtypeStruct((1, 1), jnp.float32),
        scratch_shapes=[
            pltpu.VMEM((_NBUF, _BR, _SIZE), jnp.float32),
            pltpu.SemaphoreType.DMA((_NBUF,)),
            pltpu.SMEM((1,), jnp.float32),
        ],
        compiler_params=pltpu.CompilerParams(
            dimension_semantics=("arbitrary",),
        ),
    )(tgt2d, r2d, x)


def kernel(x, target):
    xr = x.reshape(_N * _SIZE)
    r = _sc_gather(xr, target)
    out = _tc_reduce(x, target.reshape(_N, 1), r.reshape(_N, 1))
    return out[0, 0]


# speedup vs baseline: 1.0472x; 1.0371x over previous
import math

import jax
import jax.numpy as jnp
from jax import lax
from jax.experimental import pallas as pl
from jax.experimental.pallas import tpu as pltpu
from jax.experimental.pallas import tpu_sc as plsc

_SIZE = 100000
_N = 1024
_SMOOTHING = 0.1
_CONF = 1.0 - _SMOOTHING
_FILL = _SMOOTHING / (_SIZE - 2)
_C_ROW = (_SIZE - 2) * _FILL * math.log(_FILL) + _CONF * math.log(_CONF)

_NC = 2
_NS = 16
_LANES = 16
_NW = _NC * _NS
_B = _N // _NW

_HALF = _SIZE // 2
_NCHUNK = 2 * _B
_UNROLL = 25
_NACC = 8


def _sc_body(xr, tgt, out, tgt_v, idx_v, idx0_v, xt_v, x0_v, buf0, buf1,
             part_v, gsem, sems):
    wid = lax.axis_index("s") * _NC + lax.axis_index("c")
    base = wid * _B
    pltpu.sync_copy(tgt.at[pl.ds(base, _B)], tgt_v)

    for k in range(_B // _LANES):
        t = tgt_v[pl.ds(k * _LANES, _LANES)]
        i = base + k * _LANES + lax.iota(jnp.int32, _LANES)
        idx_v[pl.ds(k * _LANES, _LANES)] = i * _SIZE + t
        idx0_v[pl.ds(k * _LANES, _LANES)] = i * _SIZE
    pltpu.async_copy(xr.at[idx_v], xt_v, gsem).wait()
    pltpu.async_copy(xr.at[idx0_v], x0_v, gsem).wait()

    bufs = (buf0, buf1)

    def _chunk_copy(k):
        row, half = k // 2, k % 2
        off = (base + row) * _SIZE + half * _HALF
        return pltpu.make_async_copy(
            xr.at[pl.ds(off, _HALF)], bufs[k % 2], sems.at[k % 2]
        )

    _chunk_copy(0).start()
    zero = jnp.zeros((_LANES,), jnp.float32)
    accs = (zero,) * _NACC
    total_v = zero
    for k in range(_NCHUNK):
        if k + 1 < _NCHUNK:
            _chunk_copy(k + 1).start()
        _chunk_copy(k).wait()
        bref = bufs[k % 2]

        def _body(i, accs, bref=bref):
            o = i * (_UNROLL * _LANES)
            new = list(accs)
            for u in range(_UNROLL):
                v = bref[pl.ds(o + u * _LANES, _LANES)]
                new[u % _NACC] = new[u % _NACC] + v
            return tuple(new)

        accs = lax.fori_loop(0, _HALF // (_UNROLL * _LANES), _body, accs)
        if k % 2 == 1:
            row_acc = ((accs[0] + accs[1]) + (accs[2] + accs[3])) + (
                (accs[4] + accs[5]) + (accs[6] + accs[7])
            )
            row = k // 2
            t_vec = tgt_v[pl.ds((row // _LANES) * _LANES, _LANES)]
            t = t_vec[row % _LANES]
            mf = jnp.where(t != 0, jnp.float32(1.0), jnp.float32(0.0))
            total_v = total_v + mf * row_acc
            accs = (zero,) * _NACC

    rpart = jnp.zeros((_LANES,), jnp.float32)
    for k in range(_B // _LANES):
        t = tgt_v[pl.ds(k * _LANES, _LANES)]
        xt = xt_v[pl.ds(k * _LANES, _LANES)]
        x0 = x0_v[pl.ds(k * _LANES, _LANES)]
        r = jnp.where(
            t != 0,
            jnp.float32(_C_ROW)
            + jnp.float32(_FILL) * x0
            - jnp.float32(_CONF - _FILL) * xt,
            jnp.float32(0.0),
        )
        rpart = rpart + r
    part_v[...] = rpart - jnp.float32(_FILL) * total_v
    pltpu.sync_copy(part_v, out.at[wid])


def _sc_loss_partials(xr, target):
    return pl.kernel(
        _sc_body,
        out_type=jax.ShapeDtypeStruct((_NW, _LANES), jnp.float32),
        mesh=plsc.VectorSubcoreMesh(core_axis_name="c", subcore_axis_name="s"),
        scratch_types=[
            pltpu.VMEM((_B,), jnp.int32),
            pltpu.VMEM((_B,), jnp.int32),
            pltpu.VMEM((_B,), jnp.int32),
            pltpu.VMEM((_B,), jnp.float32),
            pltpu.VMEM((_B,), jnp.float32),
            pltpu.VMEM((_HALF,), jnp.float32),
            pltpu.VMEM((_HALF,), jnp.float32),
            pltpu.VMEM((_LANES,), jnp.float32),
            pltpu.SemaphoreType.DMA,
            pltpu.SemaphoreType.DMA((2,)),
        ],
    )(xr, target)


def _tc_finish_body(p_ref, out_ref):
    out_ref[...] = jnp.broadcast_to(jnp.sum(p_ref[...]), (1, 1))


def _tc_finish(partials):
    return pl.pallas_call(
        _tc_finish_body,
        out_shape=jax.ShapeDtypeStruct((1, 1), jnp.float32),
    )(partials)


def kernel(x, target):
    xr = x.reshape(_N * _SIZE)
    partials = _sc_loss_partials(xr, target)
    out = _tc_finish(partials)
    return out[0, 0]
